# Initial kernel scaffold; baseline (speedup 1.0000x reference)
import jax
import jax.numpy as jnp
from jax.experimental import pallas as pl


def kernel(x, edge_index, training, W1, b1, W2, b2, W3, b3):
    raise NotImplementedError("write your pallas kernel here")


# speedup vs baseline: 2.9507x; 2.9507x over previous
import functools

import jax
import jax.numpy as jnp
from jax import lax
from jax.experimental import pallas as pl
from jax.experimental.pallas import tpu as pltpu
from jax.experimental.pallas import tpu_sc as plsc

_N, _E, _D, _H, _C = 10000, 320000, 128, 128, 40

_NP = 10240
_CHUNK = 128
_NW = 32
_ROWS_PT = 80
_EP = _NW * _ROWS_PT * _CHUNK
_PAD_NODE = _N
_C2 = 128
_ROWS_PER_TILE = _NP // 16
_BLK = 1024

@functools.lru_cache(maxsize=None)
def _sc_mesh():
    return plsc.VectorSubcoreMesh(
        core_axis_name="c", subcore_axis_name="s", num_cores=2, num_subcores=16)


def _fill(buf, rows, width, value):
    def body(i, _):
        for k in range(width // 16):
            buf[i, pl.ds(k * 16, 16)] = jnp.full((16,), value, jnp.float32)
        return 0
    lax.fori_loop(0, rows, body, 0)


def _degree_body(i_hbm, out_hbm, idx, src, acc):
    cid = lax.axis_index("c")
    sid = lax.axis_index("s")
    zrow = sid * _ROWS_PER_TILE

    _fill(src, _CHUNK, _CHUNK, 0.0)
    def zero_acc(i, _):
        pltpu.sync_copy(src, acc.at[pl.ds(zrow + i * _CHUNK, _CHUNK)])
        return 0
    lax.fori_loop(0, _ROWS_PER_TILE // _CHUNK, zero_acc, 0)

    lane = lax.iota(jnp.int32, 16)
    one0 = jnp.where(lane == 0, 1.0, 0.0).astype(jnp.float32)
    def fill_src(i, _):
        src[i, pl.ds(0, 16)] = one0
        return 0
    lax.fori_loop(0, _CHUNK, fill_src, 0)
    plsc.subcore_barrier()

    tbase = (cid * 16 + sid) * _ROWS_PT
    pltpu.sync_copy(i_hbm.at[pl.ds(tbase, _ROWS_PT)], idx)

    def step(j, _):
        pltpu.sync_copy(src, acc.at[idx.at[j]], add=True)
        return 0
    lax.fori_loop(0, _ROWS_PT, step, 0)
    plsc.subcore_barrier()

    pltpu.sync_copy(acc.at[pl.ds(zrow, _ROWS_PER_TILE)],
                    out_hbm.at[cid, pl.ds(zrow, _ROWS_PER_TILE)])


@functools.lru_cache(maxsize=None)
def _degree_kernel():
    return pl.kernel(
        _degree_body,
        out_type=jax.ShapeDtypeStruct((2, _NP, _CHUNK), jnp.float32),
        mesh=_sc_mesh(),
        scratch_types=[
            pltpu.VMEM((_ROWS_PT, _CHUNK), jnp.int32),
            pltpu.VMEM((_CHUNK, _CHUNK), jnp.float32),
            pltpu.VMEM_SHARED((_NP, _CHUNK), jnp.float32),
        ],
    )


def _make_agg(width):
    def agg_body(table_hbm, s_hbm, r_hbm, out_hbm,
                 sidx, ridx, buf, acc, gsem):
        cid = lax.axis_index("c")
        sid = lax.axis_index("s")
        zrow = sid * _ROWS_PER_TILE

        _fill(buf, _CHUNK, width, 0.0)
        def zero_acc(i, _):
            pltpu.sync_copy(buf, acc.at[pl.ds(zrow + i * _CHUNK, _CHUNK)])
            return 0
        lax.fori_loop(0, _ROWS_PER_TILE // _CHUNK, zero_acc, 0)
        plsc.subcore_barrier()

        tbase = (cid * 16 + sid) * _ROWS_PT
        pltpu.sync_copy(s_hbm.at[pl.ds(tbase, _ROWS_PT)], sidx)
        pltpu.sync_copy(r_hbm.at[pl.ds(tbase, _ROWS_PT)], ridx)

        def step(j, _):
            pltpu.async_copy(table_hbm.at[sidx.at[j]], buf, gsem).wait()
            pltpu.sync_copy(buf, acc.at[ridx.at[j]], add=True)
            return 0
        lax.fori_loop(0, _ROWS_PT, step, 0)
        plsc.subcore_barrier()

        pltpu.sync_copy(acc.at[pl.ds(zrow, _ROWS_PER_TILE)],
                        out_hbm.at[cid, pl.ds(zrow, _ROWS_PER_TILE)])

    return pl.kernel(
        agg_body,
        out_type=jax.ShapeDtypeStruct((2, _NP, width), jnp.float32),
        mesh=_sc_mesh(),
        scratch_types=[
            pltpu.VMEM((_ROWS_PT, _CHUNK), jnp.int32),
            pltpu.VMEM((_ROWS_PT, _CHUNK), jnp.int32),
            pltpu.VMEM((_CHUNK, width), jnp.float32),
            pltpu.VMEM_SHARED((_NP, width), jnp.float32),
            pltpu.SemaphoreType.DMA,
        ],
    )


_make_agg = functools.lru_cache(maxsize=None)(_make_agg)


def _inv_sqrt_deg(d0, d1):
    deg = d0[0] + d1[0]
    return lax.rsqrt(jnp.maximum(deg[:, 0:1], 1.0))


def _mlp_body(x_ref, w1_ref, b1_ref, w2_ref, b2_ref, ds0_ref, ds1_ref, o_ref):
    h = jnp.dot(x_ref[...], w1_ref[...],
                preferred_element_type=jnp.float32) + b1_ref[...]
    h = jnp.where(h >= 0, h, 0.01 * h)
    h = jnp.dot(h, w2_ref[...],
                preferred_element_type=jnp.float32) + b2_ref[...]
    h = jnp.where(h >= 0, h, 0.01 * h)
    o_ref[...] = h * _inv_sqrt_deg(ds0_ref, ds1_ref)


def _proj_body(a0_ref, a1_ref, dr0_ref, dr1_ref, ds0_ref, ds1_ref,
               w3_ref, b3_ref, o_ref):
    nodes = (a0_ref[0] + a1_ref[0]) * _inv_sqrt_deg(dr0_ref, dr1_ref)
    z = jnp.dot(nodes, w3_ref[...],
                preferred_element_type=jnp.float32) + b3_ref[...]
    o_ref[...] = z * _inv_sqrt_deg(ds0_ref, ds1_ref)


def _softmax_body(a0_ref, a1_ref, dr0_ref, dr1_ref, o_ref):
    z = (a0_ref[0] + a1_ref[0]) * _inv_sqrt_deg(dr0_ref, dr1_ref)
    col = lax.broadcasted_iota(jnp.int32, z.shape, 1)
    valid = col < _C
    z = jnp.where(valid, z, -jnp.inf)
    m = jnp.max(z, axis=-1, keepdims=True)
    e = jnp.where(valid, jnp.exp(z - m), 0.0)
    o_ref[...] = e / jnp.sum(e, axis=-1, keepdims=True)


def _row_spec(width):
    return pl.BlockSpec((_BLK, width), lambda i: (i, 0))


def _full_spec(shape):
    ndim = len(shape)
    return pl.BlockSpec(shape, lambda i, _nd=ndim: (0,) * _nd)


def _part_spec(width, c):
    return pl.BlockSpec((1, _BLK, width), lambda i, _c=c: (_c, i, 0))


def kernel(x, edge_index, training, W1, b1, W2, b2, W3, b3):
    del training

    xp = jnp.zeros((_NP, _D), jnp.float32).at[:_N].set(x)
    pad = _EP - _E
    s2d = jnp.concatenate(
        [edge_index[0], jnp.full((pad,), _PAD_NODE, jnp.int32)]).reshape(-1, _CHUNK)
    r2d = jnp.concatenate(
        [edge_index[1], jnp.full((pad,), _PAD_NODE, jnp.int32)]).reshape(-1, _CHUNK)
    w3p = jnp.zeros((_D, _C2), jnp.float32).at[:, :_C].set(W3)
    b3p = jnp.zeros((_C2,), jnp.float32).at[:_C].set(b3)

    degs_p = _degree_kernel()(s2d)
    degr_p = _degree_kernel()(r2d)

    grid = (_NP // _BLK,)
    h1 = pl.pallas_call(
        _mlp_body,
        grid=grid,
        in_specs=[
            _row_spec(_D),
            _full_spec((_D, _H)), _full_spec((_H,)),
            _full_spec((_H, _H)), _full_spec((_H,)),
            _part_spec(_CHUNK, 0), _part_spec(_CHUNK, 1),
        ],
        out_specs=_row_spec(_H),
        out_shape=jax.ShapeDtypeStruct((_NP, _H), jnp.float32),
    )(xp, W1, b1, W2, b2, degs_p, degs_p)

    agg1 = _make_agg(_D)(h1, s2d, r2d)

    z1 = pl.pallas_call(
        _proj_body,
        grid=grid,
        in_specs=[
            _part_spec(_D, 0), _part_spec(_D, 1),
            _part_spec(_CHUNK, 0), _part_spec(_CHUNK, 1),
            _part_spec(_CHUNK, 0), _part_spec(_CHUNK, 1),
            _full_spec((_D, _C2)), _full_spec((_C2,)),
        ],
        out_specs=_row_spec(_C2),
        out_shape=jax.ShapeDtypeStruct((_NP, _C2), jnp.float32),
    )(agg1, agg1, degr_p, degr_p, degs_p, degs_p, w3p, b3p)

    agg2 = _make_agg(_C2)(z1, s2d, r2d)

    out = pl.pallas_call(
        _softmax_body,
        grid=grid,
        in_specs=[
            _part_spec(_C2, 0), _part_spec(_C2, 1),
            _part_spec(_CHUNK, 0), _part_spec(_CHUNK, 1),
        ],
        out_specs=_row_spec(_C2),
        out_shape=jax.ShapeDtypeStruct((_NP, _C2), jnp.float32),
    )(agg2, agg2, degr_p, degr_p)

    return out[:_N, :_C]


# speedup vs baseline: 3.1038x; 1.0519x over previous
import functools

import jax
import jax.numpy as jnp
from jax import lax
from jax.experimental import pallas as pl
from jax.experimental.pallas import tpu as pltpu
from jax.experimental.pallas import tpu_sc as plsc

_N, _E, _D, _H, _C = 10000, 320000, 128, 128, 40

_NP = 10240
_CHUNK = 128
_NW = 32
_ROWS_PT = 80
_EP = _NW * _ROWS_PT * _CHUNK
_PAD_NODE = _N
_C2 = 128
_ROWS_PER_TILE = _NP // 16
_BLK = 1024

@functools.lru_cache(maxsize=None)
def _sc_mesh():
    return plsc.VectorSubcoreMesh(
        core_axis_name="c", subcore_axis_name="s", num_cores=2, num_subcores=16)


def _fill(buf, rows, width, value):
    def body(i, _):
        for k in range(width // 16):
            buf[i, pl.ds(k * 16, 16)] = jnp.full((16,), value, jnp.float32)
        return 0
    lax.fori_loop(0, rows, body, 0)


def _degree_body(i_hbm, out_hbm, idx, src, acc):
    cid = lax.axis_index("c")
    sid = lax.axis_index("s")
    zrow = sid * _ROWS_PER_TILE

    _fill(src, _CHUNK, _CHUNK, 0.0)
    def zero_acc(i, _):
        pltpu.sync_copy(src, acc.at[pl.ds(zrow + i * _CHUNK, _CHUNK)])
        return 0
    lax.fori_loop(0, _ROWS_PER_TILE // _CHUNK, zero_acc, 0)

    lane = lax.iota(jnp.int32, 16)
    one0 = jnp.where(lane == 0, 1.0, 0.0).astype(jnp.float32)
    def fill_src(i, _):
        src[i, pl.ds(0, 16)] = one0
        return 0
    lax.fori_loop(0, _CHUNK, fill_src, 0)
    plsc.subcore_barrier()

    tbase = (cid * 16 + sid) * _ROWS_PT
    pltpu.sync_copy(i_hbm.at[pl.ds(tbase, _ROWS_PT)], idx)

    def step(j, _):
        pltpu.sync_copy(src, acc.at[idx.at[j]], add=True)
        return 0
    lax.fori_loop(0, _ROWS_PT, step, 0)
    plsc.subcore_barrier()

    pltpu.sync_copy(acc.at[pl.ds(zrow, _ROWS_PER_TILE)],
                    out_hbm.at[cid, pl.ds(zrow, _ROWS_PER_TILE)])


@functools.lru_cache(maxsize=None)
def _degree_kernel():
    return pl.kernel(
        _degree_body,
        out_type=jax.ShapeDtypeStruct((2, _NP, _CHUNK), jnp.float32),
        mesh=_sc_mesh(),
        scratch_types=[
            pltpu.VMEM((_ROWS_PT, _CHUNK), jnp.int32),
            pltpu.VMEM((_CHUNK, _CHUNK), jnp.float32),
            pltpu.VMEM_SHARED((_NP, _CHUNK), jnp.float32),
        ],
    )


def _make_agg(width):
    half = _ROWS_PT // 2

    def agg_body(table_hbm, s_hbm, r_hbm, out_hbm,
                 sidx, ridx, buf0, buf1, acc, g0, g1):
        cid = lax.axis_index("c")
        sid = lax.axis_index("s")
        zrow = sid * _ROWS_PER_TILE

        _fill(buf0, _CHUNK, width, 0.0)
        def zero_acc(i, _):
            pltpu.sync_copy(buf0, acc.at[pl.ds(zrow + i * _CHUNK, _CHUNK)])
            return 0
        lax.fori_loop(0, _ROWS_PER_TILE // _CHUNK, zero_acc, 0)
        plsc.subcore_barrier()

        tbase = (cid * 16 + sid) * _ROWS_PT
        for ph in range(2):
            pbase = tbase + ph * half
            pltpu.sync_copy(s_hbm.at[pl.ds(pbase, half)], sidx)
            pltpu.sync_copy(r_hbm.at[pl.ds(pbase, half)], ridx)
            pltpu.async_copy(table_hbm.at[sidx.at[0]], buf0, g0)

            def pair_step(j, _):
                c = 2 * j
                pltpu.make_async_copy(
                    table_hbm.at[sidx.at[0]], buf0, g0).wait()
                pltpu.async_copy(table_hbm.at[sidx.at[c + 1]], buf1, g1)
                pltpu.sync_copy(buf0, acc.at[ridx.at[c]], add=True)
                pltpu.make_async_copy(
                    table_hbm.at[sidx.at[0]], buf1, g1).wait()
                cn = jnp.minimum(c + 2, half - 1)
                pltpu.async_copy(table_hbm.at[sidx.at[cn]], buf0, g0)
                pltpu.sync_copy(buf1, acc.at[ridx.at[c + 1]], add=True)
                return 0
            lax.fori_loop(0, half // 2, pair_step, 0)
            pltpu.make_async_copy(table_hbm.at[sidx.at[0]], buf0, g0).wait()
        plsc.subcore_barrier()

        pltpu.sync_copy(acc.at[pl.ds(zrow, _ROWS_PER_TILE)],
                        out_hbm.at[cid, pl.ds(zrow, _ROWS_PER_TILE)])

    return pl.kernel(
        agg_body,
        out_type=jax.ShapeDtypeStruct((2, _NP, width), jnp.float32),
        mesh=_sc_mesh(),
        scratch_types=(
            [pltpu.VMEM((half, _CHUNK), jnp.int32),
             pltpu.VMEM((half, _CHUNK), jnp.int32),
             pltpu.VMEM((_CHUNK, width), jnp.float32),
             pltpu.VMEM((_CHUNK, width), jnp.float32),
             pltpu.VMEM_SHARED((_NP, width), jnp.float32),
             pltpu.SemaphoreType.DMA,
             pltpu.SemaphoreType.DMA]
        ),
    )


_make_agg = functools.lru_cache(maxsize=None)(_make_agg)


def _inv_sqrt_deg(d0, d1):
    deg = d0[0] + d1[0]
    return lax.rsqrt(jnp.maximum(deg[:, 0:1], 1.0))


def _mlp_body(x_ref, w1_ref, b1_ref, w2_ref, b2_ref, ds0_ref, ds1_ref, o_ref):
    h = jnp.dot(x_ref[...], w1_ref[...],
                preferred_element_type=jnp.float32) + b1_ref[...]
    h = jnp.where(h >= 0, h, 0.01 * h)
    h = jnp.dot(h, w2_ref[...],
                preferred_element_type=jnp.float32) + b2_ref[...]
    h = jnp.where(h >= 0, h, 0.01 * h)
    o_ref[...] = h * _inv_sqrt_deg(ds0_ref, ds1_ref)


def _proj_body(a0_ref, a1_ref, dr0_ref, dr1_ref, ds0_ref, ds1_ref,
               w3_ref, b3_ref, o_ref):
    nodes = (a0_ref[0] + a1_ref[0]) * _inv_sqrt_deg(dr0_ref, dr1_ref)
    z = jnp.dot(nodes, w3_ref[...],
                preferred_element_type=jnp.float32) + b3_ref[...]
    o_ref[...] = z * _inv_sqrt_deg(ds0_ref, ds1_ref)


def _softmax_body(a0_ref, a1_ref, dr0_ref, dr1_ref, o_ref):
    z = (a0_ref[0] + a1_ref[0]) * _inv_sqrt_deg(dr0_ref, dr1_ref)
    col = lax.broadcasted_iota(jnp.int32, z.shape, 1)
    valid = col < _C
    z = jnp.where(valid, z, -jnp.inf)
    m = jnp.max(z, axis=-1, keepdims=True)
    e = jnp.where(valid, jnp.exp(z - m), 0.0)
    o_ref[...] = e / jnp.sum(e, axis=-1, keepdims=True)


def _row_spec(width):
    return pl.BlockSpec((_BLK, width), lambda i: (i, 0))


def _full_spec(shape):
    ndim = len(shape)
    return pl.BlockSpec(shape, lambda i, _nd=ndim: (0,) * _nd)


def _part_spec(width, c):
    return pl.BlockSpec((1, _BLK, width), lambda i, _c=c: (_c, i, 0))


def kernel(x, edge_index, training, W1, b1, W2, b2, W3, b3):
    del training

    xp = jnp.zeros((_NP, _D), jnp.float32).at[:_N].set(x)
    pad = _EP - _E
    s2d = jnp.concatenate(
        [edge_index[0], jnp.full((pad,), _PAD_NODE, jnp.int32)]).reshape(-1, _CHUNK)
    r2d = jnp.concatenate(
        [edge_index[1], jnp.full((pad,), _PAD_NODE, jnp.int32)]).reshape(-1, _CHUNK)
    w3p = jnp.zeros((_D, _C2), jnp.float32).at[:, :_C].set(W3)
    b3p = jnp.zeros((_C2,), jnp.float32).at[:_C].set(b3)

    degs_p = _degree_kernel()(s2d)
    degr_p = _degree_kernel()(r2d)

    grid = (_NP // _BLK,)
    h1 = pl.pallas_call(
        _mlp_body,
        grid=grid,
        in_specs=[
            _row_spec(_D),
            _full_spec((_D, _H)), _full_spec((_H,)),
            _full_spec((_H, _H)), _full_spec((_H,)),
            _part_spec(_CHUNK, 0), _part_spec(_CHUNK, 1),
        ],
        out_specs=_row_spec(_H),
        out_shape=jax.ShapeDtypeStruct((_NP, _H), jnp.float32),
    )(xp, W1, b1, W2, b2, degs_p, degs_p)

    agg1 = _make_agg(_D)(h1, s2d, r2d)

    z1 = pl.pallas_call(
        _proj_body,
        grid=grid,
        in_specs=[
            _part_spec(_D, 0), _part_spec(_D, 1),
            _part_spec(_CHUNK, 0), _part_spec(_CHUNK, 1),
            _part_spec(_CHUNK, 0), _part_spec(_CHUNK, 1),
            _full_spec((_D, _C2)), _full_spec((_C2,)),
        ],
        out_specs=_row_spec(_C2),
        out_shape=jax.ShapeDtypeStruct((_NP, _C2), jnp.float32),
    )(agg1, agg1, degr_p, degr_p, degs_p, degs_p, w3p, b3p)

    agg2 = _make_agg(_C2)(z1, s2d, r2d)

    out = pl.pallas_call(
        _softmax_body,
        grid=grid,
        in_specs=[
            _part_spec(_C2, 0), _part_spec(_C2, 1),
            _part_spec(_CHUNK, 0), _part_spec(_CHUNK, 1),
        ],
        out_specs=_row_spec(_C2),
        out_shape=jax.ShapeDtypeStruct((_NP, _C2), jnp.float32),
    )(agg2, agg2, degr_p, degr_p)

    return out[:_N, :_C]


# speedup vs baseline: 7.7046x; 2.4823x over previous
import functools

import jax
import jax.numpy as jnp
from jax import lax
from jax.experimental import pallas as pl
from jax.experimental.pallas import tpu as pltpu
from jax.experimental.pallas import tpu_sc as plsc

_N, _E, _D, _H, _C = 10000, 320000, 128, 128, 40

_NP = 10240
_CHUNK = 128
_NW = 32
_ROWS_PT = 80
_EP = _NW * _ROWS_PT * _CHUNK
_PAD_NODE = _N
_C2 = 128
_ROWS_PER_TILE = _NP // 16
_BLK = 1024

@functools.lru_cache(maxsize=None)
def _sc_mesh():
    return plsc.VectorSubcoreMesh(
        core_axis_name="c", subcore_axis_name="s", num_cores=2, num_subcores=16)


def _fill(buf, rows, width, value):
    def body(i, _):
        for k in range(width // 16):
            buf[i, pl.ds(k * 16, 16)] = jnp.full((16,), value, jnp.float32)
        return 0
    lax.fori_loop(0, rows, body, 0)


def _degree_body(i_hbm, out_hbm, idx, src, acc):
    cid = lax.axis_index("c")
    sid = lax.axis_index("s")
    zrow = sid * _ROWS_PER_TILE

    _fill(src, _CHUNK, _CHUNK, 0.0)
    def zero_acc(i, _):
        pltpu.sync_copy(src, acc.at[pl.ds(zrow + i * _CHUNK, _CHUNK)])
        return 0
    lax.fori_loop(0, _ROWS_PER_TILE // _CHUNK, zero_acc, 0)

    lane = lax.iota(jnp.int32, 16)
    one0 = jnp.where(lane == 0, 1.0, 0.0).astype(jnp.float32)
    def fill_src(i, _):
        src[i, pl.ds(0, 16)] = one0
        return 0
    lax.fori_loop(0, _CHUNK, fill_src, 0)
    plsc.subcore_barrier()

    tbase = (cid * 16 + sid) * _ROWS_PT
    pltpu.sync_copy(i_hbm.at[pl.ds(tbase, _ROWS_PT)], idx)

    def step(j, _):
        pltpu.sync_copy(src, acc.at[idx.at[j]], add=True)
        return 0
    lax.fori_loop(0, _ROWS_PT, step, 0)
    plsc.subcore_barrier()

    pltpu.sync_copy(acc.at[pl.ds(zrow, _ROWS_PER_TILE)],
                    out_hbm.at[cid, pl.ds(zrow, _ROWS_PER_TILE)])


@functools.lru_cache(maxsize=None)
def _degree_kernel():
    return pl.kernel(
        _degree_body,
        out_type=jax.ShapeDtypeStruct((2, _NP, _CHUNK), jnp.float32),
        mesh=_sc_mesh(),
        scratch_types=[
            pltpu.VMEM((_ROWS_PT, _CHUNK), jnp.int32),
            pltpu.VMEM((_CHUNK, _CHUNK), jnp.float32),
            pltpu.VMEM_SHARED((_NP, _CHUNK), jnp.float32),
        ],
    )


def _make_agg(width):
    half = _ROWS_PT // 2

    def agg_body(table_hbm, s_hbm, r_hbm, out_hbm,
                 sidx, ridx, buf0, buf1, acc, g0, g1):
        cid = lax.axis_index("c")
        sid = lax.axis_index("s")
        zrow = sid * _ROWS_PER_TILE

        _fill(buf0, _CHUNK, width, 0.0)
        def zero_acc(i, _):
            pltpu.sync_copy(buf0, acc.at[pl.ds(zrow + i * _CHUNK, _CHUNK)])
            return 0
        lax.fori_loop(0, _ROWS_PER_TILE // _CHUNK, zero_acc, 0)
        plsc.subcore_barrier()

        tbase = (cid * 16 + sid) * _ROWS_PT
        for ph in range(2):
            pbase = tbase + ph * half
            pltpu.sync_copy(s_hbm.at[pl.ds(pbase, half)], sidx)
            pltpu.sync_copy(r_hbm.at[pl.ds(pbase, half)], ridx)
            pltpu.async_copy(table_hbm.at[sidx.at[0]], buf0, g0)

            def pair_step(j, _):
                c = 2 * j
                pltpu.make_async_copy(
                    table_hbm.at[sidx.at[0]], buf0, g0).wait()
                pltpu.async_copy(table_hbm.at[sidx.at[c + 1]], buf1, g1)
                pltpu.sync_copy(buf0, acc.at[ridx.at[c]], add=True)
                pltpu.make_async_copy(
                    table_hbm.at[sidx.at[0]], buf1, g1).wait()
                cn = jnp.minimum(c + 2, half - 1)
                pltpu.async_copy(table_hbm.at[sidx.at[cn]], buf0, g0)
                pltpu.sync_copy(buf1, acc.at[ridx.at[c + 1]], add=True)
                return 0
            lax.fori_loop(0, half // 2, pair_step, 0)
            pltpu.make_async_copy(table_hbm.at[sidx.at[0]], buf0, g0).wait()
        plsc.subcore_barrier()

        pltpu.sync_copy(acc.at[pl.ds(zrow, _ROWS_PER_TILE)],
                        out_hbm.at[cid, pl.ds(zrow, _ROWS_PER_TILE)])

    return pl.kernel(
        agg_body,
        out_type=jax.ShapeDtypeStruct((2, _NP, width), jnp.float32),
        mesh=_sc_mesh(),
        scratch_types=(
            [pltpu.VMEM((half, _CHUNK), jnp.int32),
             pltpu.VMEM((half, _CHUNK), jnp.int32),
             pltpu.VMEM((_CHUNK, width), jnp.float32),
             pltpu.VMEM((_CHUNK, width), jnp.float32),
             pltpu.VMEM_SHARED((_NP, width), jnp.float32),
             pltpu.SemaphoreType.DMA,
             pltpu.SemaphoreType.DMA]
        ),
    )


_make_agg = functools.lru_cache(maxsize=None)(_make_agg)


def _inv_sqrt_deg(d0, d1):
    deg = d0[0] + d1[0]
    return lax.rsqrt(jnp.maximum(deg[:, 0:1], 1.0))


def _mlp_body(x_ref, w1_ref, b1_ref, w2_ref, b2_ref, ds0_ref, ds1_ref, o_ref):
    h = jnp.dot(x_ref[...], w1_ref[...],
                preferred_element_type=jnp.float32) + b1_ref[...]
    h = jnp.where(h >= 0, h, 0.01 * h)
    h = jnp.dot(h, w2_ref[...],
                preferred_element_type=jnp.float32) + b2_ref[...]
    h = jnp.where(h >= 0, h, 0.01 * h)
    o_ref[...] = h * _inv_sqrt_deg(ds0_ref, ds1_ref)


def _proj_body(a0_ref, a1_ref, dr0_ref, dr1_ref, ds0_ref, ds1_ref,
               w3_ref, b3_ref, o_ref):
    nodes = (a0_ref[0] + a1_ref[0]) * _inv_sqrt_deg(dr0_ref, dr1_ref)
    z = jnp.dot(nodes, w3_ref[...],
                preferred_element_type=jnp.float32) + b3_ref[...]
    o_ref[...] = z * _inv_sqrt_deg(ds0_ref, ds1_ref)


def _softmax_body(a0_ref, a1_ref, dr0_ref, dr1_ref, o_ref):
    z = (a0_ref[0] + a1_ref[0]) * _inv_sqrt_deg(dr0_ref, dr1_ref)
    col = lax.broadcasted_iota(jnp.int32, z.shape, 1)
    valid = col < _C
    z = jnp.where(valid, z, -jnp.inf)
    m = jnp.max(z, axis=-1, keepdims=True)
    e = jnp.where(valid, jnp.exp(z - m), 0.0)
    o_ref[...] = e / jnp.sum(e, axis=-1, keepdims=True)


def _row_spec(width):
    return pl.BlockSpec((_BLK, width), lambda i: (i, 0))


def _full_spec(shape):
    ndim = len(shape)
    return pl.BlockSpec(shape, lambda i, _nd=ndim: (0,) * _nd)


def _part_spec(width, c):
    return pl.BlockSpec((1, _BLK, width), lambda i, _c=c: (_c, i, 0))


def kernel(x, edge_index, training, W1, b1, W2, b2, W3, b3):
    del training

    xp = jnp.zeros((_NP, _D), jnp.float32).at[:_N].set(x)
    pad = _EP - _E
    pad_idx = _PAD_NODE + (jnp.arange(pad, dtype=jnp.int32) % (_NP - _N))
    s2d = jnp.concatenate([edge_index[0], pad_idx]).reshape(-1, _CHUNK)
    r2d = jnp.concatenate([edge_index[1], pad_idx]).reshape(-1, _CHUNK)
    w3p = jnp.zeros((_D, _C2), jnp.float32).at[:, :_C].set(W3)
    b3p = jnp.zeros((_C2,), jnp.float32).at[:_C].set(b3)

    degs_p = _degree_kernel()(s2d)
    degr_p = _degree_kernel()(r2d)

    grid = (_NP // _BLK,)
    h1 = pl.pallas_call(
        _mlp_body,
        grid=grid,
        in_specs=[
            _row_spec(_D),
            _full_spec((_D, _H)), _full_spec((_H,)),
            _full_spec((_H, _H)), _full_spec((_H,)),
            _part_spec(_CHUNK, 0), _part_spec(_CHUNK, 1),
        ],
        out_specs=_row_spec(_H),
        out_shape=jax.ShapeDtypeStruct((_NP, _H), jnp.float32),
    )(xp, W1, b1, W2, b2, degs_p, degs_p)

    agg1 = _make_agg(_D)(h1, s2d, r2d)

    z1 = pl.pallas_call(
        _proj_body,
        grid=grid,
        in_specs=[
            _part_spec(_D, 0), _part_spec(_D, 1),
            _part_spec(_CHUNK, 0), _part_spec(_CHUNK, 1),
            _part_spec(_CHUNK, 0), _part_spec(_CHUNK, 1),
            _full_spec((_D, _C2)), _full_spec((_C2,)),
        ],
        out_specs=_row_spec(_C2),
        out_shape=jax.ShapeDtypeStruct((_NP, _C2), jnp.float32),
    )(agg1, agg1, degr_p, degr_p, degs_p, degs_p, w3p, b3p)

    agg2 = _make_agg(_C2)(z1, s2d, r2d)

    out = pl.pallas_call(
        _softmax_body,
        grid=grid,
        in_specs=[
            _part_spec(_C2, 0), _part_spec(_C2, 1),
            _part_spec(_CHUNK, 0), _part_spec(_CHUNK, 1),
        ],
        out_specs=_row_spec(_C2),
        out_shape=jax.ShapeDtypeStruct((_NP, _C2), jnp.float32),
    )(agg2, agg2, degr_p, degr_p)

    return out[:_N, :_C]


# speedup vs baseline: 9.2608x; 1.2020x over previous
import functools

import jax
import jax.numpy as jnp
from jax import lax
from jax.experimental import pallas as pl
from jax.experimental.pallas import tpu as pltpu
from jax.experimental.pallas import tpu_sc as plsc

_N, _E, _D, _H, _C = 10000, 320000, 128, 128, 40

_NP = 10240
_CHUNK = 128
_NW = 32
_ROWS_PT = 80
_EP = _NW * _ROWS_PT * _CHUNK
_PAD_NODE = _N
_C2 = 128
_ROWS_PER_TILE = _NP // 16
_BLK = 1024

@functools.lru_cache(maxsize=None)
def _sc_mesh():
    return plsc.VectorSubcoreMesh(
        core_axis_name="c", subcore_axis_name="s", num_cores=2, num_subcores=16)


def _fill(buf, rows, width, value):
    def body(i, _):
        for k in range(width // 16):
            buf[i, pl.ds(k * 16, 16)] = jnp.full((16,), value, jnp.float32)
        return 0
    lax.fori_loop(0, rows, body, 0)


_DW = 32


def _degree_body(i_hbm, out_hbm, idx, src, acc):
    cid = lax.axis_index("c")
    sid = lax.axis_index("s")
    zrow = sid * _ROWS_PER_TILE

    _fill(src, _CHUNK, _DW, 0.0)
    def zero_acc(i, _):
        pltpu.sync_copy(src, acc.at[pl.ds(zrow + i * _CHUNK, _CHUNK)])
        return 0
    lax.fori_loop(0, _ROWS_PER_TILE // _CHUNK, zero_acc, 0)

    lane = lax.iota(jnp.int32, 16)
    one0 = jnp.where(lane == 0, 1.0, 0.0).astype(jnp.float32)
    def fill_src(i, _):
        src[i, pl.ds(0, 16)] = one0
        return 0
    lax.fori_loop(0, _CHUNK, fill_src, 0)
    plsc.subcore_barrier()

    tbase = (cid * 16 + sid) * _ROWS_PT
    pltpu.sync_copy(i_hbm.at[pl.ds(tbase, _ROWS_PT)], idx)

    def step(j, _):
        pltpu.sync_copy(src, acc.at[idx.at[j]], add=True)
        return 0
    lax.fori_loop(0, _ROWS_PT, step, 0)
    plsc.subcore_barrier()

    pltpu.sync_copy(acc.at[pl.ds(zrow, _ROWS_PER_TILE)],
                    out_hbm.at[cid, pl.ds(zrow, _ROWS_PER_TILE)])


@functools.lru_cache(maxsize=None)
def _degree_kernel():
    return pl.kernel(
        _degree_body,
        out_type=jax.ShapeDtypeStruct((2, _NP, _DW), jnp.float32),
        mesh=_sc_mesh(),
        scratch_types=[
            pltpu.VMEM((_ROWS_PT, _CHUNK), jnp.int32),
            pltpu.VMEM((_CHUNK, _DW), jnp.float32),
            pltpu.VMEM_SHARED((_NP, _DW), jnp.float32),
        ],
    )


def _make_agg(width):
    half = _ROWS_PT // 2

    def agg_body(table_hbm, s_hbm, r_hbm, out_hbm,
                 sidx, ridx, buf0, buf1, acc, g0, g1):
        cid = lax.axis_index("c")
        sid = lax.axis_index("s")
        zrow = sid * _ROWS_PER_TILE

        _fill(buf0, _CHUNK, width, 0.0)
        def zero_acc(i, _):
            pltpu.sync_copy(buf0, acc.at[pl.ds(zrow + i * _CHUNK, _CHUNK)])
            return 0
        lax.fori_loop(0, _ROWS_PER_TILE // _CHUNK, zero_acc, 0)
        plsc.subcore_barrier()

        tbase = (cid * 16 + sid) * _ROWS_PT
        for ph in range(2):
            pbase = tbase + ph * half
            pltpu.sync_copy(s_hbm.at[pl.ds(pbase, half)], sidx)
            pltpu.sync_copy(r_hbm.at[pl.ds(pbase, half)], ridx)
            pltpu.async_copy(table_hbm.at[sidx.at[0]], buf0, g0)

            def pair_step(j, _):
                c = 2 * j
                pltpu.make_async_copy(
                    table_hbm.at[sidx.at[0]], buf0, g0).wait()
                pltpu.async_copy(table_hbm.at[sidx.at[c + 1]], buf1, g1)
                pltpu.sync_copy(buf0, acc.at[ridx.at[c]], add=True)
                pltpu.make_async_copy(
                    table_hbm.at[sidx.at[0]], buf1, g1).wait()
                cn = jnp.minimum(c + 2, half - 1)
                pltpu.async_copy(table_hbm.at[sidx.at[cn]], buf0, g0)
                pltpu.sync_copy(buf1, acc.at[ridx.at[c + 1]], add=True)
                return 0
            lax.fori_loop(0, half // 2, pair_step, 0)
            pltpu.make_async_copy(table_hbm.at[sidx.at[0]], buf0, g0).wait()
        plsc.subcore_barrier()

        pltpu.sync_copy(acc.at[pl.ds(zrow, _ROWS_PER_TILE)],
                        out_hbm.at[cid, pl.ds(zrow, _ROWS_PER_TILE)])

    return pl.kernel(
        agg_body,
        out_type=jax.ShapeDtypeStruct((2, _NP, width), jnp.float32),
        mesh=_sc_mesh(),
        scratch_types=(
            [pltpu.VMEM((half, _CHUNK), jnp.int32),
             pltpu.VMEM((half, _CHUNK), jnp.int32),
             pltpu.VMEM((_CHUNK, width), jnp.float32),
             pltpu.VMEM((_CHUNK, width), jnp.float32),
             pltpu.VMEM_SHARED((_NP, width), jnp.float32),
             pltpu.SemaphoreType.DMA,
             pltpu.SemaphoreType.DMA]
        ),
    )


_make_agg = functools.lru_cache(maxsize=None)(_make_agg)


def _inv_sqrt_deg(d0, d1):
    deg = d0[0] + d1[0]
    return lax.rsqrt(jnp.maximum(deg[:, 0:1], 1.0))


def _mlp_body(x_ref, w1_ref, b1_ref, w2_ref, b2_ref, ds0_ref, ds1_ref, o_ref):
    h = jnp.dot(x_ref[...], w1_ref[...],
                preferred_element_type=jnp.float32) + b1_ref[...]
    h = jnp.where(h >= 0, h, 0.01 * h)
    h = jnp.dot(h, w2_ref[...],
                preferred_element_type=jnp.float32) + b2_ref[...]
    h = jnp.where(h >= 0, h, 0.01 * h)
    o_ref[...] = h * _inv_sqrt_deg(ds0_ref, ds1_ref)


def _proj_body(a0_ref, a1_ref, dr0_ref, dr1_ref, ds0_ref, ds1_ref,
               w3_ref, b3_ref, o_ref):
    nodes = (a0_ref[0] + a1_ref[0]) * _inv_sqrt_deg(dr0_ref, dr1_ref)
    z = jnp.dot(nodes, w3_ref[...],
                preferred_element_type=jnp.float32) + b3_ref[...]
    o_ref[...] = z * _inv_sqrt_deg(ds0_ref, ds1_ref)


def _softmax_body(a0_ref, a1_ref, dr0_ref, dr1_ref, o_ref):
    z = (a0_ref[0] + a1_ref[0]) * _inv_sqrt_deg(dr0_ref, dr1_ref)
    col = lax.broadcasted_iota(jnp.int32, z.shape, 1)
    valid = col < _C
    z = jnp.where(valid, z, -jnp.inf)
    m = jnp.max(z, axis=-1, keepdims=True)
    e = jnp.where(valid, jnp.exp(z - m), 0.0)
    p = e / jnp.sum(e, axis=-1, keepdims=True)
    o_ref[...] = p[:, :_C]


def _row_spec(width):
    return pl.BlockSpec((_BLK, width), lambda i: (i, 0))


def _full_spec(shape):
    ndim = len(shape)
    return pl.BlockSpec(shape, lambda i, _nd=ndim: (0,) * _nd)


def _part_spec(width, c):
    return pl.BlockSpec((1, _BLK, width), lambda i, _c=c: (_c, i, 0))


def kernel(x, edge_index, training, W1, b1, W2, b2, W3, b3):
    del training

    pad = _EP - _E
    pad_idx = _PAD_NODE + (jnp.arange(pad, dtype=jnp.int32) % (_NP - _N))
    s2d = jnp.concatenate([edge_index[0], pad_idx]).reshape(-1, _CHUNK)
    r2d = jnp.concatenate([edge_index[1], pad_idx]).reshape(-1, _CHUNK)
    w3p = jnp.zeros((_D, _C2), jnp.float32).at[:, :_C].set(W3)
    b3p = jnp.zeros((_C2,), jnp.float32).at[:_C].set(b3)

    degs_p = _degree_kernel()(s2d)
    degr_p = _degree_kernel()(r2d)

    grid = (_NP // _BLK,)
    h1 = pl.pallas_call(
        _mlp_body,
        grid=grid,
        in_specs=[
            _row_spec(_D),
            _full_spec((_D, _H)), _full_spec((_H,)),
            _full_spec((_H, _H)), _full_spec((_H,)),
            _part_spec(_DW, 0), _part_spec(_DW, 1),
        ],
        out_specs=_row_spec(_H),
        out_shape=jax.ShapeDtypeStruct((_NP, _H), jnp.float32),
    )(x, W1, b1, W2, b2, degs_p, degs_p)

    agg1 = _make_agg(_D)(h1, s2d, r2d)

    z1 = pl.pallas_call(
        _proj_body,
        grid=grid,
        in_specs=[
            _part_spec(_D, 0), _part_spec(_D, 1),
            _part_spec(_DW, 0), _part_spec(_DW, 1),
            _part_spec(_DW, 0), _part_spec(_DW, 1),
            _full_spec((_D, _C2)), _full_spec((_C2,)),
        ],
        out_specs=_row_spec(_C2),
        out_shape=jax.ShapeDtypeStruct((_NP, _C2), jnp.float32),
    )(agg1, agg1, degr_p, degr_p, degs_p, degs_p, w3p, b3p)

    agg2 = _make_agg(_C2)(z1, s2d, r2d)

    out = pl.pallas_call(
        _softmax_body,
        grid=grid,
        in_specs=[
            _part_spec(_C2, 0), _part_spec(_C2, 1),
            _part_spec(_DW, 0), _part_spec(_DW, 1),
        ],
        out_specs=pl.BlockSpec((_BLK, _C), lambda i: (i, 0)),
        out_shape=jax.ShapeDtypeStruct((_N, _C), jnp.float32),
    )(agg2, agg2, degr_p, degr_p)

    return out
